# baseline (speedup 1.0000x reference)
import functools

import jax
import jax.numpy as jnp
from jax import lax
from jax.experimental import pallas as pl
from jax.experimental.pallas import tpu as pltpu
from jax.experimental.pallas import tpu_sc as plsc

B = 16384
NFLD = 12
F_IDX = 9
F_CONT = 3
NC = 2
NS = 16
L = 16
NW = NC * NS
BPW = B // NW
GROUPS = BPW // L

_mesh = plsc.VectorSubcoreMesh(core_axis_name="c", subcore_axis_name="s")


@functools.partial(
    pl.kernel,
    mesh=_mesh,
    compiler_params=pltpu.CompilerParams(needs_layout_passes=False),
    out_type=jax.ShapeDtypeStruct((B,), jnp.float32),
    scratch_types=[
        pltpu.VMEM((1, NFLD * BPW), jnp.int32),
        pltpu.VMEM((1, F_IDX * BPW), jnp.float32),
    pltpu.VMEM((BPW,), jnp.float32),
        pltpu.VMEM((L,), jnp.float32),
        pltpu.SemaphoreType.DMA,
        [pltpu.SemaphoreType.DMA] * NFLD,
    ],
)
def _fl_kernel(xt_hbm, table_hbm, bias_hbm, out_hbm,
               xv, vals_v, out_v, bias_v, sem, row_sems):
    wid = lax.axis_index("s") * NC + lax.axis_index("c")
    base = wid * BPW
    row_copies = [
        pltpu.async_copy(xt_hbm.at[pl.ds(f, 1), pl.ds(base, BPW)],
                         xv.at[:, pl.ds(f * BPW, BPW)], row_sems[f])
        for f in range(NFLD)
    ]
    copies = []
    for f in range(F_IDX):
        row_copies[f].wait()
        copies.append(
            pltpu.async_copy(table_hbm.at[xv.at[:, pl.ds(f * BPW, BPW)]],
                             vals_v.at[:, pl.ds(f * BPW, BPW)], sem))
    pltpu.sync_copy(bias_hbm, bias_v.at[pl.ds(0, 1)])
    for f in range(F_IDX, NFLD):
        row_copies[f].wait()
    lanes = lax.iota(jnp.int32, L)
    bv = plsc.load_gather(bias_v, [lanes * 0])
    for c in copies:
        c.wait()
    for g in range(GROUPS):
        o = g * L
        acc = bv
        for f in range(6):
            acc = acc + vals_v[0, pl.ds(f * BPW + o, L)]
        for k in range(F_CONT):
            acc = acc + (vals_v[0, pl.ds((6 + k) * BPW + o, L)]
                         * xv[0, pl.ds((F_IDX + k) * BPW + o, L)].astype(jnp.float32))
        out_v[pl.ds(o, L)] = acc
    pltpu.sync_copy(out_v, out_hbm.at[pl.ds(base, BPW)])


def kernel(x, fc_weight, bias):
    xt = x.astype(jnp.int32).T
    out = _fl_kernel(xt, fc_weight.astype(jnp.float32).T,
                     bias.astype(jnp.float32))
    return out.reshape(B, 1)


# speedup vs baseline: 1.0158x; 1.0158x over previous
import functools

import jax
import jax.numpy as jnp
from jax import lax
from jax.experimental import pallas as pl
from jax.experimental.pallas import tpu as pltpu
from jax.experimental.pallas import tpu_sc as plsc

B = 16384
NFLD = 12
F_IDX = 9
F_CONT = 3
NC = 2
NS = 16
L = 16
NW = NC * NS
BPW = B // NW
GROUPS = BPW // L

_mesh = plsc.VectorSubcoreMesh(core_axis_name="c", subcore_axis_name="s")


@functools.partial(
    pl.kernel,
    mesh=_mesh,
    compiler_params=pltpu.CompilerParams(needs_layout_passes=False),
    out_type=jax.ShapeDtypeStruct((B,), jnp.float32),
    scratch_types=[
        pltpu.VMEM((1, NFLD * BPW), jnp.int32),
        pltpu.VMEM((1, F_IDX * BPW), jnp.float32),
    pltpu.VMEM((BPW,), jnp.float32),
        pltpu.VMEM((L,), jnp.float32),
        pltpu.SemaphoreType.DMA,
        pltpu.SemaphoreType.DMA,
        [pltpu.SemaphoreType.DMA] * NFLD,
    ],
)
def _fl_kernel(xt_hbm, table_hbm, bias_hbm, out_hbm,
               xv, vals_v, out_v, bias_v, sem, sem_b, row_sems):
    wid = lax.axis_index("s") * NC + lax.axis_index("c")
    base = wid * BPW
    row_copies = [
        pltpu.async_copy(xt_hbm.at[pl.ds(f, 1), pl.ds(base, BPW)],
                         xv.at[:, pl.ds(f * BPW, BPW)], row_sems[f])
        for f in range(NFLD)
    ]
    copies = []
    for f in range(F_IDX):
        row_copies[f].wait()
        copies.append(
            pltpu.async_copy(table_hbm.at[xv.at[:, pl.ds(f * BPW, BPW)]],
                             vals_v.at[:, pl.ds(f * BPW, BPW)],
                             sem if f < 6 else sem_b))
    pltpu.sync_copy(bias_hbm, bias_v.at[pl.ds(0, 1)])
    lanes = lax.iota(jnp.int32, L)
    bv = plsc.load_gather(bias_v, [lanes * 0])
    for c in copies[:6]:
        c.wait()
    for g in range(GROUPS):
        o = g * L
        acc = bv
        for f in range(6):
            acc = acc + vals_v[0, pl.ds(f * BPW + o, L)]
        out_v[pl.ds(o, L)] = acc
    for f in range(F_IDX, NFLD):
        row_copies[f].wait()
    for c in copies[6:]:
        c.wait()
    for g in range(GROUPS):
        o = g * L
        acc = out_v[pl.ds(o, L)]
        for k in range(F_CONT):
            acc = acc + (vals_v[0, pl.ds((6 + k) * BPW + o, L)]
                         * xv[0, pl.ds((F_IDX + k) * BPW + o, L)].astype(jnp.float32))
        out_v[pl.ds(o, L)] = acc
    pltpu.sync_copy(out_v, out_hbm.at[pl.ds(base, BPW)])


def kernel(x, fc_weight, bias):
    xt = x.astype(jnp.int32).T
    out = _fl_kernel(xt, fc_weight.astype(jnp.float32).T,
                     bias.astype(jnp.float32))
    return out.reshape(B, 1)


# speedup vs baseline: 1.0271x; 1.0111x over previous
import functools

import jax
import jax.numpy as jnp
from jax import lax
from jax.experimental import pallas as pl
from jax.experimental.pallas import tpu as pltpu
from jax.experimental.pallas import tpu_sc as plsc

B = 16384
NFLD = 12
F_IDX = 9
F_CONT = 3
NC = 2
NS = 16
L = 16
NW = NC * NS
BPW = B // NW
GROUPS = BPW // L

_mesh = plsc.VectorSubcoreMesh(core_axis_name="c", subcore_axis_name="s")


@functools.partial(
    pl.kernel,
    mesh=_mesh,
    compiler_params=pltpu.CompilerParams(needs_layout_passes=False),
    out_type=jax.ShapeDtypeStruct((B,), jnp.float32),
    scratch_types=[
        pltpu.VMEM((1, NFLD * BPW), jnp.int32),
        pltpu.VMEM((1, F_IDX * BPW), jnp.float32),
    pltpu.VMEM((BPW,), jnp.float32),
        pltpu.VMEM((L,), jnp.float32),
        pltpu.SemaphoreType.DMA,
        pltpu.SemaphoreType.DMA,
        [pltpu.SemaphoreType.DMA] * NFLD,
    ],
)
def _fl_kernel(xt_hbm, table_hbm, bias_hbm, out_hbm,
               xv, vals_v, out_v, bias_v, sem, sem_b, row_sems):
    wid = lax.axis_index("s") * NC + lax.axis_index("c")
    base = wid * BPW
    row_copies = [
        pltpu.async_copy(xt_hbm.at[pl.ds(f, 1), pl.ds(base, BPW)],
                         xv.at[:, pl.ds(f * BPW, BPW)], row_sems[f])
        for f in range(NFLD)
    ]
    copies = []
    for f in range(F_IDX):
        row_copies[f].wait()
        copies.append(
            pltpu.async_copy(table_hbm.at[xv.at[:, pl.ds(f * BPW, BPW)]],
                             vals_v.at[:, pl.ds(f * BPW, BPW)],
                             sem if f < 6 else sem_b))
    pltpu.sync_copy(bias_hbm, bias_v.at[pl.ds(0, 1)])
    lanes = lax.iota(jnp.int32, L)
    bv = plsc.load_gather(bias_v, [lanes * 0])
    for c in copies[:6]:
        c.wait()
    for g in range(GROUPS):
        o = g * L
        acc = bv
        for f in range(6):
            acc = acc + vals_v[0, pl.ds(f * BPW + o, L)]
        out_v[pl.ds(o, L)] = acc
    for f in range(F_IDX, NFLD):
        row_copies[f].wait()
    for c in copies[6:]:
        c.wait()
    out_copies = []
    half = BPW // 2
    for h in range(2):
        for g in range(GROUPS // 2):
            o = h * half + g * L
            acc = out_v[pl.ds(o, L)]
            for k in range(F_CONT):
                acc = acc + (vals_v[0, pl.ds((6 + k) * BPW + o, L)]
                             * xv[0, pl.ds((F_IDX + k) * BPW + o, L)].astype(jnp.float32))
            out_v[pl.ds(o, L)] = acc
        out_copies.append(
            pltpu.async_copy(out_v.at[pl.ds(h * half, half)],
                             out_hbm.at[pl.ds(base + h * half, half)], sem_b))
    for c in out_copies:
        c.wait()


def kernel(x, fc_weight, bias):
    xt = x.astype(jnp.int32).T
    out = _fl_kernel(xt, fc_weight.astype(jnp.float32).T,
                     bias.astype(jnp.float32))
    return out.reshape(B, 1)
